# Initial kernel scaffold; baseline (speedup 1.0000x reference)
import jax
import jax.numpy as jnp
from jax.experimental import pallas as pl


def kernel(z_e_x, embedding):
    raise NotImplementedError("write your pallas kernel here")


# speedup vs baseline: 1.2812x; 1.2812x over previous
import jax
import jax.numpy as jnp
from jax.experimental import pallas as pl

K, D = 1024, 256
BM = 2048


def _vq_kernel(x_ref, et_ref, out_ref):
    x = x_ref[...]
    et = et_ref[...]
    codebook_sqr = jnp.sum(et * et, axis=0, keepdims=True)
    inputs_sqr = jnp.sum(x * x, axis=1, keepdims=True)
    mm = jnp.dot(x, et, preferred_element_type=jnp.float32)
    distances = (codebook_sqr + inputs_sqr) - 2.0 * mm
    dmin = jnp.min(distances, axis=1, keepdims=True)
    lane = jax.lax.broadcasted_iota(jnp.int32, (BM, K), 1)
    idx = jnp.min(jnp.where(distances == dmin, lane, K),
                  axis=1, keepdims=True)
    out_ref[...] = idx


def kernel(z_e_x, embedding):
    B, T, d = z_e_x.shape
    M = B * T
    x = z_e_x.reshape(M, d)
    et = embedding.T
    out = pl.pallas_call(
        _vq_kernel,
        grid=(M // BM,),
        in_specs=[
            pl.BlockSpec((BM, D), lambda i: (i, 0)),
            pl.BlockSpec((D, K), lambda i: (0, 0)),
        ],
        out_specs=pl.BlockSpec((BM, 1), lambda i: (i, 0)),
        out_shape=jax.ShapeDtypeStruct((M, 1), jnp.int32),
    )(x, et)
    return out.reshape(B, T)
